# initial kernel scaffold (unmeasured)
import jax
import jax.numpy as jnp
from jax import lax
from jax.experimental import pallas as pl
from jax.experimental.pallas import tpu as pltpu

NZ = 4
T = 4096
D = 2048
V_SHARD = 8192
CH = T // NZ


def _allreduce_z(partial):

    def body(p_hbm, out_ref, recv_buf, copy_sem, send_sems, recv_rs, recv_ag):
        my_x = lax.axis_index("x")
        my_y = lax.axis_index("y")
        my_z = lax.axis_index("z")
        left = lax.rem(my_z + NZ - 1, NZ)
        right = lax.rem(my_z + 1, NZ)

        cp = pltpu.make_async_copy(p_hbm, out_ref, copy_sem)
        cp.start()
        cp.wait()

        barrier = pltpu.get_barrier_semaphore()
        for nbr in (left, right):
            pl.semaphore_signal(
                barrier, inc=1,
                device_id=(my_x, my_y, nbr),
                device_id_type=pl.DeviceIdType.MESH,
            )
        pl.semaphore_wait(barrier, 2)

        for s in range(NZ - 1):
            c_send = lax.rem(my_z + NZ - s, NZ)
            c_recv = lax.rem(my_z + NZ - s - 1, NZ)
            rdma = pltpu.make_async_remote_copy(
                src_ref=out_ref.at[pl.ds(c_send * CH, CH), :],
                dst_ref=recv_buf.at[s],
                send_sem=send_sems.at[s],
                recv_sem=recv_rs.at[s],
                device_id=(my_x, my_y, right),
                device_id_type=pl.DeviceIdType.MESH,
            )
            rdma.start()
            rdma.wait()
            out_ref[pl.ds(c_recv * CH, CH), :] = (
                out_ref[pl.ds(c_recv * CH, CH), :] + recv_buf[s]
            )

        for s in range(NZ - 1):
            c_send = lax.rem(my_z + 1 + NZ - s, NZ)
            rdma = pltpu.make_async_remote_copy(
                src_ref=out_ref.at[pl.ds(c_send * CH, CH), :],
                dst_ref=out_ref.at[pl.ds(c_send * CH, CH), :],
                send_sem=send_sems.at[NZ - 1 + s],
                recv_sem=recv_ag.at[s],
                device_id=(my_x, my_y, right),
                device_id_type=pl.DeviceIdType.MESH,
            )
            rdma.start()
            rdma.wait()

    return pl.pallas_call(
        body,
        out_shape=jax.ShapeDtypeStruct((T, D), jnp.float32),
        in_specs=[pl.BlockSpec(memory_space=pltpu.ANY)],
        out_specs=pl.BlockSpec(memory_space=pltpu.VMEM),
        scratch_shapes=[
            pltpu.VMEM((NZ - 1, CH, D), jnp.float32),
            pltpu.SemaphoreType.DMA,
            pltpu.SemaphoreType.DMA((2 * (NZ - 1),)),
            pltpu.SemaphoreType.DMA((NZ - 1,)),
            pltpu.SemaphoreType.DMA((NZ - 1,)),
        ],
        compiler_params=pltpu.CompilerParams(collective_id=0),
    )(partial)


def kernel(ids, E):
    my_z = lax.axis_index("z")
    local = ids - my_z * V_SHARD
    valid = (local >= 0) & (local < V_SHARD)
    idx = jnp.where(valid, local, 0)
    partial = jnp.where(valid[:, None], E[idx], jnp.float32(0))
    return _allreduce_z(partial)


# baseline (device time: 3318189 ns/iter reference)
import jax
import jax.numpy as jnp
from jax import lax
from jax.experimental import pallas as pl
from jax.experimental.pallas import tpu as pltpu

NZ = 4
T = 4096
D = 2048
V_SHARD = 8192
CH = T // NZ


def _allreduce_z(partial):

    def body(p_hbm, out_ref, recv_buf, copy_sem, send_sems, recv_rs, recv_ag):
        my_x = lax.axis_index("x")
        my_y = lax.axis_index("y")
        my_z = lax.axis_index("z")
        left = lax.rem(my_z + NZ - 1, NZ)
        right = lax.rem(my_z + 1, NZ)

        cp = pltpu.make_async_copy(p_hbm, out_ref, copy_sem)
        cp.start()
        cp.wait()

        barrier = pltpu.get_barrier_semaphore()
        for nbr in (left, right):
            pl.semaphore_signal(
                barrier, inc=1,
                device_id=(my_x, my_y, nbr),
                device_id_type=pl.DeviceIdType.MESH,
            )
        pl.semaphore_wait(barrier, 2)

        for s in range(NZ - 1):
            c_send = lax.rem(my_z + NZ - s, NZ)
            c_recv = lax.rem(my_z + NZ - s - 1, NZ)
            rdma = pltpu.make_async_remote_copy(
                src_ref=out_ref.at[pl.ds(c_send * CH, CH), :],
                dst_ref=recv_buf.at[s],
                send_sem=send_sems.at[s],
                recv_sem=recv_rs.at[s],
                device_id=(my_x, my_y, right),
                device_id_type=pl.DeviceIdType.MESH,
            )
            rdma.start()
            rdma.wait()
            out_ref[pl.ds(c_recv * CH, CH), :] = (
                out_ref[pl.ds(c_recv * CH, CH), :] + recv_buf[s]
            )

        for s in range(NZ - 1):
            c_send = lax.rem(my_z + 1 + NZ - s, NZ)
            rdma = pltpu.make_async_remote_copy(
                src_ref=out_ref.at[pl.ds(c_send * CH, CH), :],
                dst_ref=out_ref.at[pl.ds(c_send * CH, CH), :],
                send_sem=send_sems.at[NZ - 1 + s],
                recv_sem=recv_ag.at[s],
                device_id=(my_x, my_y, right),
                device_id_type=pl.DeviceIdType.MESH,
            )
            rdma.start()
            rdma.wait()

    return pl.pallas_call(
        body,
        out_shape=jax.ShapeDtypeStruct((T, D), jnp.float32),
        in_specs=[pl.BlockSpec(memory_space=pl.ANY)],
        out_specs=pl.BlockSpec(memory_space=pltpu.VMEM),
        scratch_shapes=[
            pltpu.VMEM((NZ - 1, CH, D), jnp.float32),
            pltpu.SemaphoreType.DMA,
            pltpu.SemaphoreType.DMA((2 * (NZ - 1),)),
            pltpu.SemaphoreType.DMA((NZ - 1,)),
            pltpu.SemaphoreType.DMA((NZ - 1,)),
        ],
        compiler_params=pltpu.CompilerParams(
            collective_id=0,
            vmem_limit_bytes=60 * 1024 * 1024,
        ),
    )(partial)


def kernel(ids, E):
    my_z = lax.axis_index("z")
    local = ids - my_z * V_SHARD
    valid = (local >= 0) & (local < V_SHARD)
    idx = jnp.where(valid, local, 0)
    partial = jnp.where(valid[:, None], E[idx], jnp.float32(0))
    return _allreduce_z(partial)


# device time: 695725 ns/iter; 4.7694x vs baseline; 4.7694x over previous
import jax
import jax.numpy as jnp
from jax import lax
from jax.experimental import pallas as pl
from jax.experimental.pallas import tpu as pltpu

NZ = 4
T = 4096
D = 2048
V_SHARD = 8192
CH = T // NZ


def kernel(ids, E):
    def body(ids_ref, e_hbm, out_ref, recv_buf,
             gather_sem, send_sems, recv_rs, recv_ag):
        my_x = lax.axis_index("x")
        my_y = lax.axis_index("y")
        my_z = lax.axis_index("z")
        left = lax.rem(my_z + NZ - 1, NZ)
        right = lax.rem(my_z + 1, NZ)

        barrier = pltpu.get_barrier_semaphore()
        for nbr in (left, right):
            pl.semaphore_signal(
                barrier, inc=1,
                device_id=(my_x, my_y, nbr),
                device_id_type=pl.DeviceIdType.MESH,
            )
        pl.semaphore_wait(barrier, 2)

        out_ref[...] = jnp.zeros((T, D), jnp.float32)

        def issue(t, n):
            idx = ids_ref[t] - my_z * V_SHARD
            valid = jnp.logical_and(idx >= 0, idx < V_SHARD)

            @pl.when(valid)
            def _():
                pltpu.make_async_copy(
                    e_hbm.at[pl.ds(idx, 1), :],
                    out_ref.at[pl.ds(t, 1), :],
                    gather_sem,
                ).start()

            return n + valid.astype(jnp.int32)

        n_rows = lax.fori_loop(0, T, issue, jnp.int32(0))

        def drain(i, _):
            pltpu.make_async_copy(
                e_hbm.at[pl.ds(0, 1), :],
                out_ref.at[pl.ds(0, 1), :],
                gather_sem,
            ).wait()
            return 0

        lax.fori_loop(0, n_rows, drain, 0)

        for s in range(NZ - 1):
            c_send = lax.rem(my_z + NZ - s, NZ)
            c_recv = lax.rem(my_z + NZ - s - 1, NZ)
            rdma = pltpu.make_async_remote_copy(
                src_ref=out_ref.at[pl.ds(c_send * CH, CH), :],
                dst_ref=recv_buf.at[s],
                send_sem=send_sems.at[s],
                recv_sem=recv_rs.at[s],
                device_id=(my_x, my_y, right),
                device_id_type=pl.DeviceIdType.MESH,
            )
            rdma.start()
            rdma.wait()
            out_ref[pl.ds(c_recv * CH, CH), :] = (
                out_ref[pl.ds(c_recv * CH, CH), :] + recv_buf[s]
            )

        for s in range(NZ - 1):
            c_send = lax.rem(my_z + 1 + NZ - s, NZ)
            rdma = pltpu.make_async_remote_copy(
                src_ref=out_ref.at[pl.ds(c_send * CH, CH), :],
                dst_ref=out_ref.at[pl.ds(c_send * CH, CH), :],
                send_sem=send_sems.at[NZ - 1 + s],
                recv_sem=recv_ag.at[s],
                device_id=(my_x, my_y, right),
                device_id_type=pl.DeviceIdType.MESH,
            )
            rdma.start()
            rdma.wait()

    return pl.pallas_call(
        body,
        out_shape=jax.ShapeDtypeStruct((T, D), jnp.float32),
        in_specs=[
            pl.BlockSpec(memory_space=pltpu.SMEM),
            pl.BlockSpec(memory_space=pl.ANY),
        ],
        out_specs=pl.BlockSpec(memory_space=pltpu.VMEM),
        scratch_shapes=[
            pltpu.VMEM((NZ - 1, CH, D), jnp.float32),
            pltpu.SemaphoreType.DMA,
            pltpu.SemaphoreType.DMA((2 * (NZ - 1),)),
            pltpu.SemaphoreType.DMA((NZ - 1,)),
            pltpu.SemaphoreType.DMA((NZ - 1,)),
        ],
        compiler_params=pltpu.CompilerParams(
            collective_id=0,
            vmem_limit_bytes=60 * 1024 * 1024,
        ),
    )(ids, E)


# device time: 435801 ns/iter; 7.6140x vs baseline; 1.5964x over previous
import jax
import jax.numpy as jnp
from jax import lax
from jax.experimental import pallas as pl
from jax.experimental.pallas import tpu as pltpu

NZ = 4
T = 4096
D = 2048
V_SHARD = 8192
CH = T // NZ
DC = D // 4
HD = DC // 2


def kernel(ids, E):
    def body(ids_ref, e_hbm, out_ref, recv_buf,
             gather_sem, send_sems, recv_rs, recv_ag, xy_send, xy_recv):
        my_x = lax.axis_index("x")
        my_y = lax.axis_index("y")
        my_z = lax.axis_index("z")
        left = lax.rem(my_z + NZ - 1, NZ)
        right = lax.rem(my_z + 1, NZ)

        my_c = my_x * 2 + my_y
        xn_c = (1 - my_x) * 2 + my_y
        yn_c = my_x * 2 + (1 - my_y)
        dg_c = (1 - my_x) * 2 + (1 - my_y)
        dc0 = my_c * DC

        barrier = pltpu.get_barrier_semaphore()
        for nbr in (
            (my_x, my_y, left),
            (my_x, my_y, right),
            (1 - my_x, my_y, my_z),
            (my_x, 1 - my_y, my_z),
        ):
            pl.semaphore_signal(
                barrier, inc=1,
                device_id=nbr,
                device_id_type=pl.DeviceIdType.MESH,
            )
        pl.semaphore_wait(barrier, 4)

        out_ref[:, pl.ds(dc0, DC)] = jnp.zeros((T, DC), jnp.float32)

        def issue(t, n):
            idx = ids_ref[t] - my_z * V_SHARD
            valid = jnp.logical_and(idx >= 0, idx < V_SHARD)

            @pl.when(valid)
            def _():
                pltpu.make_async_copy(
                    e_hbm.at[pl.ds(idx, 1), pl.ds(dc0, DC)],
                    out_ref.at[pl.ds(t, 1), pl.ds(dc0, DC)],
                    gather_sem,
                ).start()

            return n + valid.astype(jnp.int32)

        n_rows = lax.fori_loop(0, T, issue, jnp.int32(0))

        def drain(i, _):
            pltpu.make_async_copy(
                e_hbm.at[pl.ds(0, 1), pl.ds(dc0, DC)],
                out_ref.at[pl.ds(0, 1), pl.ds(dc0, DC)],
                gather_sem,
            ).wait()
            return 0

        lax.fori_loop(0, n_rows, drain, 0)

        for s in range(NZ - 1):
            c_send = lax.rem(my_z + NZ - s, NZ)
            c_recv = lax.rem(my_z + NZ - s - 1, NZ)
            rdma = pltpu.make_async_remote_copy(
                src_ref=out_ref.at[pl.ds(c_send * CH, CH), pl.ds(dc0, DC)],
                dst_ref=recv_buf.at[s],
                send_sem=send_sems.at[s],
                recv_sem=recv_rs.at[s],
                device_id=(my_x, my_y, right),
                device_id_type=pl.DeviceIdType.MESH,
            )
            rdma.start()
            rdma.wait()
            out_ref[pl.ds(c_recv * CH, CH), pl.ds(dc0, DC)] = (
                out_ref[pl.ds(c_recv * CH, CH), pl.ds(dc0, DC)] + recv_buf[s]
            )

        for s in range(NZ - 1):
            c_send = lax.rem(my_z + 1 + NZ - s, NZ)
            sl = (pl.ds(c_send * CH, CH), pl.ds(dc0, DC))
            rdma = pltpu.make_async_remote_copy(
                src_ref=out_ref.at[sl],
                dst_ref=out_ref.at[sl],
                send_sem=send_sems.at[NZ - 1 + s],
                recv_sem=recv_ag.at[s],
                device_id=(my_x, my_y, right),
                device_id_type=pl.DeviceIdType.MESH,
            )
            rdma.start()
            rdma.wait()

        to_x = pltpu.make_async_remote_copy(
            src_ref=out_ref.at[:, pl.ds(dc0, DC)],
            dst_ref=out_ref.at[:, pl.ds(dc0, DC)],
            send_sem=xy_send.at[0],
            recv_sem=xy_recv.at[0],
            device_id=(1 - my_x, my_y, my_z),
            device_id_type=pl.DeviceIdType.MESH,
        )
        to_y = pltpu.make_async_remote_copy(
            src_ref=out_ref.at[:, pl.ds(dc0, DC)],
            dst_ref=out_ref.at[:, pl.ds(dc0, DC)],
            send_sem=xy_send.at[1],
            recv_sem=xy_recv.at[1],
            device_id=(my_x, 1 - my_y, my_z),
            device_id_type=pl.DeviceIdType.MESH,
        )
        to_x.start()
        to_y.start()
        to_x.wait()
        to_y.wait()

        fwd_x = pltpu.make_async_remote_copy(
            src_ref=out_ref.at[:, pl.ds(yn_c * DC, HD)],
            dst_ref=out_ref.at[:, pl.ds(yn_c * DC, HD)],
            send_sem=xy_send.at[2],
            recv_sem=xy_recv.at[2],
            device_id=(1 - my_x, my_y, my_z),
            device_id_type=pl.DeviceIdType.MESH,
        )
        fwd_y = pltpu.make_async_remote_copy(
            src_ref=out_ref.at[:, pl.ds(xn_c * DC + HD, HD)],
            dst_ref=out_ref.at[:, pl.ds(xn_c * DC + HD, HD)],
            send_sem=xy_send.at[3],
            recv_sem=xy_recv.at[3],
            device_id=(my_x, 1 - my_y, my_z),
            device_id_type=pl.DeviceIdType.MESH,
        )
        fwd_x.start()
        fwd_y.start()
        fwd_x.wait()
        fwd_y.wait()

    return pl.pallas_call(
        body,
        out_shape=jax.ShapeDtypeStruct((T, D), jnp.float32),
        in_specs=[
            pl.BlockSpec(memory_space=pltpu.SMEM),
            pl.BlockSpec(memory_space=pl.ANY),
        ],
        out_specs=pl.BlockSpec(memory_space=pltpu.VMEM),
        scratch_shapes=[
            pltpu.VMEM((NZ - 1, CH, DC), jnp.float32),
            pltpu.SemaphoreType.DMA,
            pltpu.SemaphoreType.DMA((2 * (NZ - 1),)),
            pltpu.SemaphoreType.DMA((NZ - 1,)),
            pltpu.SemaphoreType.DMA((NZ - 1,)),
            pltpu.SemaphoreType.DMA((4,)),
            pltpu.SemaphoreType.DMA((4,)),
        ],
        compiler_params=pltpu.CompilerParams(
            collective_id=0,
            vmem_limit_bytes=60 * 1024 * 1024,
        ),
    )(ids, E)


# device time: 345562 ns/iter; 9.6023x vs baseline; 1.2611x over previous
import jax
import jax.numpy as jnp
from jax import lax
from jax.experimental import pallas as pl
from jax.experimental.pallas import tpu as pltpu

NZ = 4
T = 4096
D = 2048
V_SHARD = 8192
CH = T // NZ
DC = D // 4
HD = DC // 2


def kernel(ids, E):
    def body(ids_ref, e_hbm, out_ref, recv_buf,
             gather_sem, send_sems, recv_rs, recv_ag, xy_send, xy_recv):
        my_x = lax.axis_index("x")
        my_y = lax.axis_index("y")
        my_z = lax.axis_index("z")
        left = lax.rem(my_z + NZ - 1, NZ)
        right = lax.rem(my_z + 1, NZ)

        my_c = my_x * 2 + my_y
        xn_c = (1 - my_x) * 2 + my_y
        yn_c = my_x * 2 + (1 - my_y)
        dg_c = (1 - my_x) * 2 + (1 - my_y)
        dc0 = my_c * DC

        barrier = pltpu.get_barrier_semaphore()
        for nbr in (
            (my_x, my_y, left),
            (my_x, my_y, right),
            (1 - my_x, my_y, my_z),
            (my_x, 1 - my_y, my_z),
        ):
            pl.semaphore_signal(
                barrier, inc=1,
                device_id=nbr,
                device_id_type=pl.DeviceIdType.MESH,
            )

        out_ref[:, pl.ds(dc0, DC)] = jnp.zeros((T, DC), jnp.float32)

        def gather_chunk(k):

            def issue(i, n):
                t = k * CH + i
                idx = ids_ref[t] - my_z * V_SHARD
                valid = jnp.logical_and(idx >= 0, idx < V_SHARD)

                @pl.when(valid)
                def _():
                    pltpu.make_async_copy(
                        e_hbm.at[pl.ds(idx, 1), pl.ds(dc0, DC)],
                        out_ref.at[pl.ds(t, 1), pl.ds(dc0, DC)],
                        gather_sem,
                    ).start()

                return n + valid.astype(jnp.int32)

            return lax.fori_loop(0, CH, issue, jnp.int32(0), unroll=8)

        def drain_chunk(n_rows):
            def drain(i, _):
                pltpu.make_async_copy(
                    e_hbm.at[pl.ds(0, 1), pl.ds(dc0, DC)],
                    out_ref.at[pl.ds(0, 1), pl.ds(dc0, DC)],
                    gather_sem,
                ).wait()
                return 0

            lax.fori_loop(0, n_rows, drain, 0)

        drain_chunk(gather_chunk(my_z))
        pl.semaphore_wait(barrier, 4)

        for s in range(NZ - 1):
            c_send = lax.rem(my_z + NZ - s, NZ)
            c_recv = lax.rem(my_z + NZ - s - 1, NZ)
            rdma = pltpu.make_async_remote_copy(
                src_ref=out_ref.at[pl.ds(c_send * CH, CH), pl.ds(dc0, DC)],
                dst_ref=recv_buf.at[s],
                send_sem=send_sems.at[s],
                recv_sem=recv_rs.at[s],
                device_id=(my_x, my_y, right),
                device_id_type=pl.DeviceIdType.MESH,
            )
            rdma.start()
            n = gather_chunk(c_recv)
            drain_chunk(n)
            rdma.wait()
            out_ref[pl.ds(c_recv * CH, CH), pl.ds(dc0, DC)] = (
                out_ref[pl.ds(c_recv * CH, CH), pl.ds(dc0, DC)] + recv_buf[s]
            )

        for s in range(NZ - 1):
            c_send = lax.rem(my_z + 1 + NZ - s, NZ)
            sl = (pl.ds(c_send * CH, CH), pl.ds(dc0, DC))
            rdma = pltpu.make_async_remote_copy(
                src_ref=out_ref.at[sl],
                dst_ref=out_ref.at[sl],
                send_sem=send_sems.at[NZ - 1 + s],
                recv_sem=recv_ag.at[s],
                device_id=(my_x, my_y, right),
                device_id_type=pl.DeviceIdType.MESH,
            )
            rdma.start()
            rdma.wait()

        to_x = pltpu.make_async_remote_copy(
            src_ref=out_ref.at[:, pl.ds(dc0, DC)],
            dst_ref=out_ref.at[:, pl.ds(dc0, DC)],
            send_sem=xy_send.at[0],
            recv_sem=xy_recv.at[0],
            device_id=(1 - my_x, my_y, my_z),
            device_id_type=pl.DeviceIdType.MESH,
        )
        to_y = pltpu.make_async_remote_copy(
            src_ref=out_ref.at[:, pl.ds(dc0, DC)],
            dst_ref=out_ref.at[:, pl.ds(dc0, DC)],
            send_sem=xy_send.at[1],
            recv_sem=xy_recv.at[1],
            device_id=(my_x, 1 - my_y, my_z),
            device_id_type=pl.DeviceIdType.MESH,
        )
        to_x.start()
        to_y.start()
        to_x.wait()
        to_y.wait()

        fwd_x = pltpu.make_async_remote_copy(
            src_ref=out_ref.at[:, pl.ds(yn_c * DC, HD)],
            dst_ref=out_ref.at[:, pl.ds(yn_c * DC, HD)],
            send_sem=xy_send.at[2],
            recv_sem=xy_recv.at[2],
            device_id=(1 - my_x, my_y, my_z),
            device_id_type=pl.DeviceIdType.MESH,
        )
        fwd_y = pltpu.make_async_remote_copy(
            src_ref=out_ref.at[:, pl.ds(xn_c * DC + HD, HD)],
            dst_ref=out_ref.at[:, pl.ds(xn_c * DC + HD, HD)],
            send_sem=xy_send.at[3],
            recv_sem=xy_recv.at[3],
            device_id=(my_x, 1 - my_y, my_z),
            device_id_type=pl.DeviceIdType.MESH,
        )
        fwd_x.start()
        fwd_y.start()
        fwd_x.wait()
        fwd_y.wait()

    return pl.pallas_call(
        body,
        out_shape=jax.ShapeDtypeStruct((T, D), jnp.float32),
        in_specs=[
            pl.BlockSpec(memory_space=pltpu.SMEM),
            pl.BlockSpec(memory_space=pl.ANY),
        ],
        out_specs=pl.BlockSpec(memory_space=pltpu.VMEM),
        scratch_shapes=[
            pltpu.VMEM((NZ - 1, CH, DC), jnp.float32),
            pltpu.SemaphoreType.DMA,
            pltpu.SemaphoreType.DMA((2 * (NZ - 1),)),
            pltpu.SemaphoreType.DMA((NZ - 1,)),
            pltpu.SemaphoreType.DMA((NZ - 1,)),
            pltpu.SemaphoreType.DMA((4,)),
            pltpu.SemaphoreType.DMA((4,)),
        ],
        compiler_params=pltpu.CompilerParams(
            collective_id=0,
            vmem_limit_bytes=60 * 1024 * 1024,
        ),
    )(ids, E)


# device time: 275988 ns/iter; 12.0229x vs baseline; 1.2521x over previous
import jax
import jax.numpy as jnp
from jax import lax
from jax.experimental import pallas as pl
from jax.experimental.pallas import tpu as pltpu

NZ = 4
T = 4096
D = 2048
V_SHARD = 8192
CH = T // NZ
DC = D // 4
HD = DC // 2


def kernel(ids, E):
    def body(ids_ref, e_hbm, out_ref, recv_buf,
             gather_sem, send_sems, recv_rs, recv_ag, xy_send, xy_recv):
        my_x = lax.axis_index("x")
        my_y = lax.axis_index("y")
        my_z = lax.axis_index("z")
        left = lax.rem(my_z + NZ - 1, NZ)
        right = lax.rem(my_z + 1, NZ)

        my_c = my_x * 2 + my_y
        xn_c = (1 - my_x) * 2 + my_y
        yn_c = my_x * 2 + (1 - my_y)
        dg_c = (1 - my_x) * 2 + (1 - my_y)
        dc0 = my_c * DC

        barrier = pltpu.get_barrier_semaphore()
        for nbr in (
            (my_x, my_y, left),
            (my_x, my_y, right),
            (1 - my_x, my_y, my_z),
            (my_x, 1 - my_y, my_z),
        ):
            pl.semaphore_signal(
                barrier, inc=1,
                device_id=nbr,
                device_id_type=pl.DeviceIdType.MESH,
            )

        out_ref[:, pl.ds(dc0, DC)] = jnp.zeros((T, DC), jnp.float32)

        def gather_chunk(k):

            def issue(i, n):
                t = k * CH + i
                idx = ids_ref[t] - my_z * V_SHARD
                valid = jnp.logical_and(idx >= 0, idx < V_SHARD)

                @pl.when(valid)
                def _():
                    pltpu.make_async_copy(
                        e_hbm.at[pl.ds(idx, 1), pl.ds(dc0, DC)],
                        out_ref.at[pl.ds(t, 1), pl.ds(dc0, DC)],
                        gather_sem,
                    ).start()

                return n + valid.astype(jnp.int32)

            return lax.fori_loop(0, CH, issue, jnp.int32(0), unroll=8)

        def drain_chunk(n_rows):
            def drain(i, _):
                pltpu.make_async_copy(
                    e_hbm.at[pl.ds(0, 1), pl.ds(dc0, DC)],
                    out_ref.at[pl.ds(0, 1), pl.ds(dc0, DC)],
                    gather_sem,
                ).wait()
                return 0

            lax.fori_loop(0, n_rows, drain, 0)

        drain_chunk(gather_chunk(my_z))
        pl.semaphore_wait(barrier, 4)

        for s in range(NZ - 1):
            c_send = lax.rem(my_z + NZ - s, NZ)
            c_recv = lax.rem(my_z + NZ - s - 1, NZ)
            rdma = pltpu.make_async_remote_copy(
                src_ref=out_ref.at[pl.ds(c_send * CH, CH), pl.ds(dc0, DC)],
                dst_ref=recv_buf.at[s],
                send_sem=send_sems.at[s],
                recv_sem=recv_rs.at[s],
                device_id=(my_x, my_y, right),
                device_id_type=pl.DeviceIdType.MESH,
            )
            rdma.start()
            n = gather_chunk(c_recv)
            drain_chunk(n)
            rdma.wait()
            out_ref[pl.ds(c_recv * CH, CH), pl.ds(dc0, DC)] = (
                out_ref[pl.ds(c_recv * CH, CH), pl.ds(dc0, DC)] + recv_buf[s]
            )

        def xy1(j, k):
            sl = (pl.ds(k * CH, CH), pl.ds(dc0, DC))
            to_x = pltpu.make_async_remote_copy(
                src_ref=out_ref.at[sl],
                dst_ref=out_ref.at[sl],
                send_sem=xy_send.at[j],
                recv_sem=xy_recv.at[j],
                device_id=(1 - my_x, my_y, my_z),
                device_id_type=pl.DeviceIdType.MESH,
            )
            to_y = pltpu.make_async_remote_copy(
                src_ref=out_ref.at[sl],
                dst_ref=out_ref.at[sl],
                send_sem=xy_send.at[NZ + j],
                recv_sem=xy_recv.at[NZ + j],
                device_id=(my_x, 1 - my_y, my_z),
                device_id_type=pl.DeviceIdType.MESH,
            )
            return to_x, to_y

        def xy2(j, k):
            sl_x = (pl.ds(k * CH, CH), pl.ds(yn_c * DC, HD))
            sl_y = (pl.ds(k * CH, CH), pl.ds(xn_c * DC + HD, HD))
            fwd_x = pltpu.make_async_remote_copy(
                src_ref=out_ref.at[sl_x],
                dst_ref=out_ref.at[sl_x],
                send_sem=xy_send.at[2 * NZ + j],
                recv_sem=xy_recv.at[2 * NZ + j],
                device_id=(1 - my_x, my_y, my_z),
                device_id_type=pl.DeviceIdType.MESH,
            )
            fwd_y = pltpu.make_async_remote_copy(
                src_ref=out_ref.at[sl_y],
                dst_ref=out_ref.at[sl_y],
                send_sem=xy_send.at[3 * NZ + j],
                recv_sem=xy_recv.at[3 * NZ + j],
                device_id=(my_x, 1 - my_y, my_z),
                device_id_type=pl.DeviceIdType.MESH,
            )
            return fwd_x, fwd_y

        txs, tys, fws = [], [], []
        k0 = lax.rem(my_z + 1, NZ)
        to_x, to_y = xy1(0, k0)
        to_x.start()
        to_y.start()
        txs.append(to_x)
        tys.append(to_y)

        for s in range(NZ - 1):
            k_s = lax.rem(my_z + 1 + NZ - s, NZ)
            sl = (pl.ds(k_s * CH, CH), pl.ds(dc0, DC))
            ag = pltpu.make_async_remote_copy(
                src_ref=out_ref.at[sl],
                dst_ref=out_ref.at[sl],
                send_sem=send_sems.at[NZ - 1 + s],
                recv_sem=recv_ag.at[s],
                device_id=(my_x, my_y, right),
                device_id_type=pl.DeviceIdType.MESH,
            )
            ag.start()
            txs[s].wait()
            tys[s].wait()
            fwd_x, fwd_y = xy2(s, k_s)
            fwd_x.start()
            fwd_y.start()
            fws.append((fwd_x, fwd_y))
            ag.wait()
            k_next = lax.rem(my_z + NZ - s, NZ)
            to_x, to_y = xy1(s + 1, k_next)
            to_x.start()
            to_y.start()
            txs.append(to_x)
            tys.append(to_y)

        txs[NZ - 1].wait()
        tys[NZ - 1].wait()
        k_last = lax.rem(my_z + 2, NZ)
        fwd_x, fwd_y = xy2(NZ - 1, k_last)
        fwd_x.start()
        fwd_y.start()
        fws.append((fwd_x, fwd_y))
        for fwd_x, fwd_y in fws:
            fwd_x.wait()
            fwd_y.wait()

    return pl.pallas_call(
        body,
        out_shape=jax.ShapeDtypeStruct((T, D), jnp.float32),
        in_specs=[
            pl.BlockSpec(memory_space=pltpu.SMEM),
            pl.BlockSpec(memory_space=pl.ANY),
        ],
        out_specs=pl.BlockSpec(memory_space=pltpu.VMEM),
        scratch_shapes=[
            pltpu.VMEM((NZ - 1, CH, DC), jnp.float32),
            pltpu.SemaphoreType.DMA,
            pltpu.SemaphoreType.DMA((2 * (NZ - 1),)),
            pltpu.SemaphoreType.DMA((NZ - 1,)),
            pltpu.SemaphoreType.DMA((NZ - 1,)),
            pltpu.SemaphoreType.DMA((4 * NZ,)),
            pltpu.SemaphoreType.DMA((4 * NZ,)),
        ],
        compiler_params=pltpu.CompilerParams(
            collective_id=0,
            vmem_limit_bytes=60 * 1024 * 1024,
        ),
    )(ids, E)


# device time: 270513 ns/iter; 12.2663x vs baseline; 1.0202x over previous
import jax
import jax.numpy as jnp
from jax import lax
from jax.experimental import pallas as pl
from jax.experimental.pallas import tpu as pltpu

NZ = 4
T = 4096
D = 2048
V_SHARD = 8192
CH = T // NZ
DC = D // 4
HD = DC // 2


def kernel(ids, E):
    def body(ids_ref, e_hbm, out_ref, recv_buf,
             gather_sem, send_sems, recv_rs, recv_ag, xy_send, xy_recv):
        my_x = lax.axis_index("x")
        my_y = lax.axis_index("y")
        my_z = lax.axis_index("z")
        left = lax.rem(my_z + NZ - 1, NZ)
        right = lax.rem(my_z + 1, NZ)

        my_c = my_x * 2 + my_y
        xn_c = (1 - my_x) * 2 + my_y
        yn_c = my_x * 2 + (1 - my_y)
        dg_c = (1 - my_x) * 2 + (1 - my_y)
        dc0 = my_c * DC

        barrier = pltpu.get_barrier_semaphore()
        for nbr in (
            (my_x, my_y, left),
            (my_x, my_y, right),
            (1 - my_x, my_y, my_z),
            (my_x, 1 - my_y, my_z),
        ):
            pl.semaphore_signal(
                barrier, inc=1,
                device_id=nbr,
                device_id_type=pl.DeviceIdType.MESH,
            )

        def gather_chunk(k):
            out_ref[pl.ds(k * CH, CH), pl.ds(dc0, DC)] = jnp.zeros(
                (CH, DC), jnp.float32
            )

            def issue(i, n):
                t = k * CH + i
                idx = ids_ref[t] - my_z * V_SHARD
                valid = jnp.logical_and(idx >= 0, idx < V_SHARD)

                @pl.when(valid)
                def _():
                    pltpu.make_async_copy(
                        e_hbm.at[pl.ds(idx, 1), pl.ds(dc0, DC)],
                        out_ref.at[pl.ds(t, 1), pl.ds(dc0, DC)],
                        gather_sem,
                    ).start()

                return n + valid.astype(jnp.int32)

            return lax.fori_loop(0, CH, issue, jnp.int32(0), unroll=8)

        def drain_chunk(n_rows):
            def drain(i, _):
                pltpu.make_async_copy(
                    e_hbm.at[pl.ds(0, 1), pl.ds(dc0, DC)],
                    out_ref.at[pl.ds(0, 1), pl.ds(dc0, DC)],
                    gather_sem,
                ).wait()
                return 0

            lax.fori_loop(0, n_rows, drain, 0)

        drain_chunk(gather_chunk(my_z))
        pl.semaphore_wait(barrier, 4)

        for s in range(NZ - 1):
            c_send = lax.rem(my_z + NZ - s, NZ)
            c_recv = lax.rem(my_z + NZ - s - 1, NZ)
            rdma = pltpu.make_async_remote_copy(
                src_ref=out_ref.at[pl.ds(c_send * CH, CH), pl.ds(dc0, DC)],
                dst_ref=recv_buf.at[s],
                send_sem=send_sems.at[s],
                recv_sem=recv_rs.at[s],
                device_id=(my_x, my_y, right),
                device_id_type=pl.DeviceIdType.MESH,
            )
            rdma.start()
            n = gather_chunk(c_recv)
            drain_chunk(n)
            rdma.wait()
            out_ref[pl.ds(c_recv * CH, CH), pl.ds(dc0, DC)] = (
                out_ref[pl.ds(c_recv * CH, CH), pl.ds(dc0, DC)] + recv_buf[s]
            )

        def xy1(j, k):
            sl = (pl.ds(k * CH, CH), pl.ds(dc0, DC))
            to_x = pltpu.make_async_remote_copy(
                src_ref=out_ref.at[sl],
                dst_ref=out_ref.at[sl],
                send_sem=xy_send.at[j],
                recv_sem=xy_recv.at[j],
                device_id=(1 - my_x, my_y, my_z),
                device_id_type=pl.DeviceIdType.MESH,
            )
            to_y = pltpu.make_async_remote_copy(
                src_ref=out_ref.at[sl],
                dst_ref=out_ref.at[sl],
                send_sem=xy_send.at[NZ + j],
                recv_sem=xy_recv.at[NZ + j],
                device_id=(my_x, 1 - my_y, my_z),
                device_id_type=pl.DeviceIdType.MESH,
            )
            return to_x, to_y

        def xy2(j, k):
            sl_x = (pl.ds(k * CH, CH), pl.ds(yn_c * DC, HD))
            sl_y = (pl.ds(k * CH, CH), pl.ds(xn_c * DC + HD, HD))
            fwd_x = pltpu.make_async_remote_copy(
                src_ref=out_ref.at[sl_x],
                dst_ref=out_ref.at[sl_x],
                send_sem=xy_send.at[2 * NZ + j],
                recv_sem=xy_recv.at[2 * NZ + j],
                device_id=(1 - my_x, my_y, my_z),
                device_id_type=pl.DeviceIdType.MESH,
            )
            fwd_y = pltpu.make_async_remote_copy(
                src_ref=out_ref.at[sl_y],
                dst_ref=out_ref.at[sl_y],
                send_sem=xy_send.at[3 * NZ + j],
                recv_sem=xy_recv.at[3 * NZ + j],
                device_id=(my_x, 1 - my_y, my_z),
                device_id_type=pl.DeviceIdType.MESH,
            )
            return fwd_x, fwd_y

        def mk_ag(s):
            k_s = lax.rem(my_z + 1 + NZ - s, NZ)
            sl = (pl.ds(k_s * CH, CH), pl.ds(dc0, DC))
            return pltpu.make_async_remote_copy(
                src_ref=out_ref.at[sl],
                dst_ref=out_ref.at[sl],
                send_sem=send_sems.at[NZ - 1 + s],
                recv_sem=recv_ag.at[s],
                device_id=(my_x, my_y, right),
                device_id_type=pl.DeviceIdType.MESH,
            )

        txs, tys, fws = [], [], []
        k0 = lax.rem(my_z + 1, NZ)
        to_x, to_y = xy1(0, k0)
        to_x.start()
        to_y.start()
        txs.append(to_x)
        tys.append(to_y)
        ag = mk_ag(0)
        ag.start()

        for s in range(NZ - 1):
            ag.wait()
            if s + 1 < NZ - 1:
                ag = mk_ag(s + 1)
                ag.start()
            k_next = lax.rem(my_z + NZ - s, NZ)
            to_x, to_y = xy1(s + 1, k_next)
            to_x.start()
            to_y.start()
            txs.append(to_x)
            tys.append(to_y)
            k_s = lax.rem(my_z + 1 + NZ - s, NZ)
            txs[s].wait()
            tys[s].wait()
            fwd_x, fwd_y = xy2(s, k_s)
            fwd_x.start()
            fwd_y.start()
            fws.append((fwd_x, fwd_y))

        txs[NZ - 1].wait()
        tys[NZ - 1].wait()
        k_last = lax.rem(my_z + 2, NZ)
        fwd_x, fwd_y = xy2(NZ - 1, k_last)
        fwd_x.start()
        fwd_y.start()
        fws.append((fwd_x, fwd_y))
        for fwd_x, fwd_y in fws:
            fwd_x.wait()
            fwd_y.wait()

    return pl.pallas_call(
        body,
        out_shape=jax.ShapeDtypeStruct((T, D), jnp.float32),
        in_specs=[
            pl.BlockSpec(memory_space=pltpu.SMEM),
            pl.BlockSpec(memory_space=pl.ANY),
        ],
        out_specs=pl.BlockSpec(memory_space=pltpu.VMEM),
        scratch_shapes=[
            pltpu.VMEM((NZ - 1, CH, DC), jnp.float32),
            pltpu.SemaphoreType.DMA,
            pltpu.SemaphoreType.DMA((2 * (NZ - 1),)),
            pltpu.SemaphoreType.DMA((NZ - 1,)),
            pltpu.SemaphoreType.DMA((NZ - 1,)),
            pltpu.SemaphoreType.DMA((4 * NZ,)),
            pltpu.SemaphoreType.DMA((4 * NZ,)),
        ],
        compiler_params=pltpu.CompilerParams(
            collective_id=0,
            vmem_limit_bytes=60 * 1024 * 1024,
        ),
    )(ids, E)


# device time: 267404 ns/iter; 12.4089x vs baseline; 1.0116x over previous
import jax
import jax.numpy as jnp
from jax import lax
from jax.experimental import pallas as pl
from jax.experimental.pallas import tpu as pltpu

NZ = 4
T = 4096
D = 2048
V_SHARD = 8192
CH = T // NZ
DC = D // 4
HD = DC // 2
QD = HD // 2


def kernel(ids, E):
    def body(ids_ref, e_hbm, out_ref, recv_buf,
             gather_sem, send_sems, recv_rs, recv_ag, xy_send, xy_recv):
        my_x = lax.axis_index("x")
        my_y = lax.axis_index("y")
        my_z = lax.axis_index("z")
        left = lax.rem(my_z + NZ - 1, NZ)
        right = lax.rem(my_z + 1, NZ)

        my_c = my_x * 2 + my_y
        xn_c = (1 - my_x) * 2 + my_y
        yn_c = my_x * 2 + (1 - my_y)
        dg_c = (1 - my_x) * 2 + (1 - my_y)
        dc0 = my_c * DC

        barrier = pltpu.get_barrier_semaphore()
        for nbr in (
            (my_x, my_y, left),
            (my_x, my_y, right),
            (1 - my_x, my_y, my_z),
            (my_x, 1 - my_y, my_z),
        ):
            pl.semaphore_signal(
                barrier, inc=1,
                device_id=nbr,
                device_id_type=pl.DeviceIdType.MESH,
            )

        def gather_chunk(k):
            out_ref[pl.ds(k * CH, CH), pl.ds(dc0, DC)] = jnp.zeros(
                (CH, DC), jnp.float32
            )

            def issue(i, n):
                t = k * CH + i
                idx = ids_ref[t] - my_z * V_SHARD
                valid = jnp.logical_and(idx >= 0, idx < V_SHARD)

                @pl.when(valid)
                def _():
                    pltpu.make_async_copy(
                        e_hbm.at[pl.ds(idx, 1), pl.ds(dc0, DC)],
                        out_ref.at[pl.ds(t, 1), pl.ds(dc0, DC)],
                        gather_sem,
                    ).start()

                return n + valid.astype(jnp.int32)

            return lax.fori_loop(0, CH, issue, jnp.int32(0), unroll=8)

        def drain_chunk(n_rows):
            def drain(i, _):
                pltpu.make_async_copy(
                    e_hbm.at[pl.ds(0, 1), pl.ds(dc0, DC)],
                    out_ref.at[pl.ds(0, 1), pl.ds(dc0, DC)],
                    gather_sem,
                ).wait()
                return 0

            lax.fori_loop(0, n_rows, drain, 0)

        drain_chunk(gather_chunk(my_z))
        pl.semaphore_wait(barrier, 4)

        for s in range(NZ - 1):
            c_send = lax.rem(my_z + NZ - s, NZ)
            c_recv = lax.rem(my_z + NZ - s - 1, NZ)
            rdma = pltpu.make_async_remote_copy(
                src_ref=out_ref.at[pl.ds(c_send * CH, CH), pl.ds(dc0, DC)],
                dst_ref=recv_buf.at[s],
                send_sem=send_sems.at[s],
                recv_sem=recv_rs.at[s],
                device_id=(my_x, my_y, right),
                device_id_type=pl.DeviceIdType.MESH,
            )
            rdma.start()
            n = gather_chunk(c_recv)
            drain_chunk(n)
            rdma.wait()
            out_ref[pl.ds(c_recv * CH, CH), pl.ds(dc0, DC)] = (
                out_ref[pl.ds(c_recv * CH, CH), pl.ds(dc0, DC)] + recv_buf[s]
            )

        def xy1(j, k):
            sl = (pl.ds(k * CH, CH), pl.ds(dc0, DC))
            to_x = pltpu.make_async_remote_copy(
                src_ref=out_ref.at[sl],
                dst_ref=out_ref.at[sl],
                send_sem=xy_send.at[j],
                recv_sem=xy_recv.at[j],
                device_id=(1 - my_x, my_y, my_z),
                device_id_type=pl.DeviceIdType.MESH,
            )
            to_y = pltpu.make_async_remote_copy(
                src_ref=out_ref.at[sl],
                dst_ref=out_ref.at[sl],
                send_sem=xy_send.at[NZ + j],
                recv_sem=xy_recv.at[NZ + j],
                device_id=(my_x, 1 - my_y, my_z),
                device_id_type=pl.DeviceIdType.MESH,
            )
            return to_x, to_y

        par = lax.rem(my_z, 2)

        def xy2(j, k):
            sl_x = (pl.ds(k * CH, CH), pl.ds(yn_c * DC + par * HD, QD))
            sl_y = (pl.ds(k * CH, CH), pl.ds(xn_c * DC + par * HD + QD, QD))
            fwd_x = pltpu.make_async_remote_copy(
                src_ref=out_ref.at[sl_x],
                dst_ref=out_ref.at[sl_x],
                send_sem=xy_send.at[2 * NZ + j],
                recv_sem=xy_recv.at[2 * NZ + j],
                device_id=(1 - my_x, my_y, my_z),
                device_id_type=pl.DeviceIdType.MESH,
            )
            fwd_y = pltpu.make_async_remote_copy(
                src_ref=out_ref.at[sl_y],
                dst_ref=out_ref.at[sl_y],
                send_sem=xy_send.at[3 * NZ + j],
                recv_sem=xy_recv.at[3 * NZ + j],
                device_id=(my_x, 1 - my_y, my_z),
                device_id_type=pl.DeviceIdType.MESH,
            )
            return fwd_x, fwd_y

        def pair_xchg(j, k):
            sl = (pl.ds(k * CH, CH), pl.ds(dg_c * DC + par * HD, HD))
            return pltpu.make_async_remote_copy(
                src_ref=out_ref.at[sl],
                dst_ref=out_ref.at[sl],
                send_sem=xy_send.at[4 * NZ + j],
                recv_sem=xy_recv.at[4 * NZ + j],
                device_id=(my_x, my_y, my_z - 2 * par + 1),
                device_id_type=pl.DeviceIdType.MESH,
            )

        def mk_ag(s):
            k_s = lax.rem(my_z + 1 + NZ - s, NZ)
            sl = (pl.ds(k_s * CH, CH), pl.ds(dc0, DC))
            return pltpu.make_async_remote_copy(
                src_ref=out_ref.at[sl],
                dst_ref=out_ref.at[sl],
                send_sem=send_sems.at[NZ - 1 + s],
                recv_sem=recv_ag.at[s],
                device_id=(my_x, my_y, right),
                device_id_type=pl.DeviceIdType.MESH,
            )

        txs, tys, fws, pairs = [], [], [], []
        k0 = lax.rem(my_z + 1, NZ)
        to_x, to_y = xy1(0, k0)
        to_x.start()
        to_y.start()
        txs.append(to_x)
        tys.append(to_y)
        ag = mk_ag(0)
        ag.start()

        for s in range(NZ - 1):
            ag.wait()
            if s + 1 < NZ - 1:
                ag = mk_ag(s + 1)
                ag.start()
            k_next = lax.rem(my_z + NZ - s, NZ)
            to_x, to_y = xy1(s + 1, k_next)
            to_x.start()
            to_y.start()
            txs.append(to_x)
            tys.append(to_y)
            k_s = lax.rem(my_z + 1 + NZ - s, NZ)
            txs[s].wait()
            tys[s].wait()
            fwd_x, fwd_y = xy2(s, k_s)
            fwd_x.start()
            fwd_y.start()
            fws.append((fwd_x, fwd_y, k_s))
            if s >= 1:
                pfx, pfy, pk = fws[s - 1]
                pfx.wait()
                pfy.wait()
                pair = pair_xchg(s - 1, pk)
                pair.start()
                pairs.append(pair)

        txs[NZ - 1].wait()
        tys[NZ - 1].wait()
        k_last = lax.rem(my_z + 2, NZ)
        fwd_x, fwd_y = xy2(NZ - 1, k_last)
        fwd_x.start()
        fwd_y.start()
        fws.append((fwd_x, fwd_y, k_last))
        for j in range(NZ - 2, NZ):
            pfx, pfy, pk = fws[j]
            pfx.wait()
            pfy.wait()
            pair = pair_xchg(j, pk)
            pair.start()
            pairs.append(pair)
        for pair in pairs:
            pair.wait()

    return pl.pallas_call(
        body,
        out_shape=jax.ShapeDtypeStruct((T, D), jnp.float32),
        in_specs=[
            pl.BlockSpec(memory_space=pltpu.SMEM),
            pl.BlockSpec(memory_space=pl.ANY),
        ],
        out_specs=pl.BlockSpec(memory_space=pltpu.VMEM),
        scratch_shapes=[
            pltpu.VMEM((NZ - 1, CH, DC), jnp.float32),
            pltpu.SemaphoreType.DMA,
            pltpu.SemaphoreType.DMA((2 * (NZ - 1),)),
            pltpu.SemaphoreType.DMA((NZ - 1,)),
            pltpu.SemaphoreType.DMA((NZ - 1,)),
            pltpu.SemaphoreType.DMA((5 * NZ,)),
            pltpu.SemaphoreType.DMA((5 * NZ,)),
        ],
        compiler_params=pltpu.CompilerParams(
            collective_id=0,
            vmem_limit_bytes=60 * 1024 * 1024,
        ),
    )(ids, E)
